# baseline (speedup 1.0000x reference)
import jax
import jax.numpy as jnp
from jax.experimental import pallas as pl

F32 = jnp.float32
BF16 = jnp.bfloat16
I32 = jnp.int32


def _dot(a, b):
    return jnp.dot(a, b, preferred_element_type=F32)


def _oh_lanes(idx_col, val, rows, cols):
    ii = jax.lax.broadcasted_iota(I32, (rows, cols), 1)
    return jnp.where(ii == idx_col, val, jnp.zeros((), val.dtype))


def _oh_rows(idx_row, rows, cols):
    ii = jax.lax.broadcasted_iota(I32, (rows, cols), 0)
    return jnp.where(ii == idx_row, jnp.ones((), F32), jnp.zeros((), F32))


def _layer(x, r, src_ref, dst_ref, et_ref, normc, wi, wo, wl, wr, aff_ref,
           li):
    n_ent, _ = x.shape
    n_rel2 = r.shape[0]
    e2 = src_ref.shape[0]
    e_h = e2 // 2

    h = _dot(_oh_lanes(src_ref[...], normc, e2, n_ent), x)
    re = _dot(_oh_lanes(et_ref[...], jnp.ones((), F32), e2, n_rel2), r)
    m = h * re
    msg_in = _dot(m[:e_h], wi)
    msg_out = _dot(m[e_h:], wo)
    agg = _dot(_oh_rows(dst_ref[:, :e_h], n_ent, e_h), msg_in)
    agg = agg + _dot(_oh_rows(dst_ref[:, e_h:], n_ent, e_h), msg_out)
    lr = aff_ref[3 * li:3 * li + 1]
    loopm = _dot(x * lr, wl)
    scale = aff_ref[3 * li + 1:3 * li + 2]
    shift = aff_ref[3 * li + 2:3 * li + 3]
    x_next = (agg + loopm) * scale + shift
    r_next = _dot(r, wr)
    return x_next, r_next


def _l0_kernel(x_ref, r_ref, src_ref, dst_ref, et_ref, norm_ref,
               wi_ref, wo_ref, wl_ref, wr_ref, aff_ref,
               x_out_ref, r_out_ref):
    x1, r1 = _layer(x_ref[...], r_ref[...],
                    src_ref, dst_ref, et_ref, norm_ref[...],
                    wi_ref[...], wo_ref[...], wl_ref[...], wr_ref[...],
                    aff_ref, 0)
    x_out_ref[...] = x1.astype(BF16)
    r_out_ref[...] = r1.astype(BF16)


def _l1_kernel(x_ref, r_ref, src_ref, dst_ref, et_ref, norm_ref,
               wi_ref, wo_ref, wl_ref, wr_ref, aff_ref,
               subj_ref, rel_ref,
               x_out_ref, sub_ref, rel_out_ref):
    n_ent = x_ref.shape[0]
    n_rel2 = r_ref.shape[0]
    batch = subj_ref.shape[0]
    x2, r2 = _layer(x_ref[...].astype(F32), r_ref[...].astype(F32),
                    src_ref, dst_ref, et_ref, norm_ref[...],
                    wi_ref[...], wo_ref[...], wl_ref[...], wr_ref[...],
                    aff_ref, 1)
    x_out_ref[...] = x2
    one = jnp.ones((), F32)
    sub_ref[...] = _dot(_oh_lanes(subj_ref[...], one, batch, n_ent),
                        x2)
    rel_out_ref[...] = _dot(_oh_lanes(rel_ref[...], one, batch, n_rel2),
                            r2)


def _affine(bias, gamma, beta, mean, var, eps=1e-5):
    scale = gamma * jax.lax.rsqrt(var + eps)
    shift = (bias - mean) * scale + beta
    return scale * (1.0 / 3.0), shift


def _full_specs(ops):
    return [pl.BlockSpec(op.shape, lambda n=op.ndim: (0,) * n) for op in ops]


def kernel(init_embed, init_rel, l0_in_w, l0_out_w, l0_loop_w, l0_w_rel,
           l0_loop_rel, l0_bias, l0_bn_gamma, l0_bn_beta, l0_bn_mean,
           l0_bn_var, l1_in_w, l1_out_w, l1_loop_w, l1_w_rel, l1_loop_rel,
           l1_bias, l1_bn_gamma, l1_bn_beta, l1_bn_mean, l1_bn_var,
           src, dst, etype, norm, subj, rel):
    n_ent, d = init_embed.shape
    r2 = init_rel.shape[0]
    e2 = src.shape[0]
    batch = subj.shape[0]

    idx = (src.reshape(e2, 1).astype(I32),
           dst.reshape(1, e2).astype(I32),
           etype.reshape(e2, 1).astype(I32),
           norm.reshape(e2, 1).astype(F32))

    scale0, shift0 = _affine(l0_bias, l0_bn_gamma, l0_bn_beta, l0_bn_mean,
                             l0_bn_var)
    scale1, shift1 = _affine(l1_bias, l1_bn_gamma, l1_bn_beta, l1_bn_mean,
                             l1_bn_var)
    aff = jnp.stack([l0_loop_rel[0], scale0, shift0,
                     l1_loop_rel[0], scale1, shift1])

    ops0 = (init_embed, init_rel) + idx + (l0_in_w, l0_out_w, l0_loop_w,
                                           l0_w_rel, aff)
    x1, r1 = pl.pallas_call(
        _l0_kernel,
        in_specs=_full_specs(ops0),
        out_specs=(pl.BlockSpec((n_ent, d), lambda: (0, 0)),
                   pl.BlockSpec((r2, d), lambda: (0, 0))),
        out_shape=(jax.ShapeDtypeStruct((n_ent, d), BF16),
                   jax.ShapeDtypeStruct((r2, d), BF16)),
    )(*ops0)

    subj_c = subj.reshape(batch, 1).astype(I32)
    rel_c = rel.reshape(batch, 1).astype(I32)
    ops1 = (x1, r1) + idx + (l1_in_w, l1_out_w, l1_loop_w, l1_w_rel, aff,
                             subj_c, rel_c)
    x2, sub_emb, rel_emb = pl.pallas_call(
        _l1_kernel,
        in_specs=_full_specs(ops1),
        out_specs=(pl.BlockSpec((n_ent, d), lambda: (0, 0)),
                   pl.BlockSpec((batch, d), lambda: (0, 0)),
                   pl.BlockSpec((batch, d), lambda: (0, 0))),
        out_shape=(jax.ShapeDtypeStruct((n_ent, d), F32),
                   jax.ShapeDtypeStruct((batch, d), F32),
                   jax.ShapeDtypeStruct((batch, d), F32)),
    )(*ops1)
    return sub_emb, rel_emb, x2


# speedup vs baseline: 1.0052x; 1.0052x over previous
import jax
import jax.numpy as jnp
from jax.experimental import pallas as pl
from jax.experimental.pallas import tpu as pltpu

F32 = jnp.float32
BF16 = jnp.bfloat16


def _onehot_rows(n_rows, idx_lanes, dtype):
    ii = jax.lax.broadcasted_iota(jnp.int32, (n_rows, idx_lanes.shape[1]), 0)
    return (ii == idx_lanes).astype(dtype)


def _ta_dot(a, b):
    return jax.lax.dot_general(a, b, (((0,), (0,)), ((), ())),
                               preferred_element_type=F32)


def _dot(a, b):
    return jnp.dot(a, b, preferred_element_type=F32)


def _direction_agg(xb, rb, g_idx, s_idx, et_loc, norm, wb, n_ent):
    gather = _onehot_rows(n_ent, g_idx, BF16)
    h = _ta_dot(gather, xb)
    rel_oh = _onehot_rows(rb.shape[0], et_loc, BF16)
    r_edge = _ta_dot(rel_oh, rb)
    mb = (h * r_edge).astype(BF16)
    msg = _dot(mb, wb).astype(BF16)
    n_iota = jax.lax.broadcasted_iota(jnp.int32, (n_ent, s_idx.shape[1]), 0)
    scat = ((n_iota == s_idx).astype(F32) * norm).astype(BF16)
    return _dot(scat, msg)


def _layer_tail(d, xb, rb, looprel_ref, loopw_ref, wrel_ref,
                loop_ref, rout_ref):
    nh = xb.shape[0] // 2
    lr = looprel_ref[...].astype(BF16)
    lwb = loopw_ref[...].astype(BF16)

    @pl.when(d == 0)
    def _():
        loop_ref[...] = _dot(xb[:nh] * lr, lwb).astype(BF16)

    @pl.when(d == 1)
    def _():
        loop_ref[...] = _dot(xb[nh:] * lr, lwb).astype(BF16)

    rout_ref[...] = _dot(rb, wrel_ref[...].astype(BF16))


def _layer0_kernel(x_ref, r_ref, src_ref, dst_ref, et_ref, norm_ref,
                   inw_ref, outw_ref, loopw_ref, wrel_ref, looprel_ref,
                   agg_ref, loop_ref, rout_ref):
    d = pl.program_id(0)
    n_ent = x_ref.shape[0]
    xb = x_ref[...].astype(BF16)
    rb = r_ref[...].astype(BF16)
    wb = jnp.where(d == 0, inw_ref[...], outw_ref[...]).astype(BF16)
    et_loc = et_ref[...] - d * r_ref.shape[0]
    agg = _direction_agg(xb, rb, src_ref[...], dst_ref[...], et_loc,
                         norm_ref[...], wb, n_ent)
    agg_ref[...] = agg[None].astype(BF16)
    _layer_tail(d, xb, rb, looprel_ref, loopw_ref, wrel_ref,
                loop_ref, rout_ref)


def _layer1_kernel(aggp_ref, loopp_ref, scale_ref, shift_ref,
                   r_ref, src_ref, dst_ref, et_ref, norm_ref,
                   inw_ref, outw_ref, loopw_ref, wrel_ref, looprel_ref,
                   agg_ref, loop_ref, rout_ref):
    d = pl.program_id(0)
    n_ent = loopp_ref.shape[0]
    x1 = (aggp_ref[0].astype(F32) + aggp_ref[1].astype(F32)
          + loopp_ref[...].astype(F32)) * scale_ref[...] + shift_ref[...]
    xb = x1.astype(BF16)
    rb = r_ref[...].astype(BF16)
    wb = jnp.where(d == 0, inw_ref[...], outw_ref[...]).astype(BF16)
    et_loc = et_ref[...] - d * r_ref.shape[0]
    agg = _direction_agg(xb, rb, src_ref[...], dst_ref[...], et_loc,
                         norm_ref[...], wb, n_ent)
    agg_ref[...] = agg[None].astype(BF16)
    _layer_tail(d, xb, rb, looprel_ref, loopw_ref, wrel_ref,
                loop_ref, rout_ref)


def _final_kernel(agg_ref, loop_ref, scale_ref, shift_ref, r2_ref,
                  subj_ref, rel_ref,
                  x_ref, subp_ref, relp_ref):
    d = pl.program_id(0)
    nh = loop_ref.shape[0]
    x2 = ((agg_ref[0].astype(F32) + agg_ref[1].astype(F32)
           + loop_ref[...].astype(F32)) * scale_ref[...] + shift_ref[...])
    x_ref[...] = x2
    sj = subj_ref[...] - d * nh
    subp_ref[...] = _ta_dot(_onehot_rows(nh, sj, F32), x2)[None]
    rl = rel_ref[...] - d * r2_ref.shape[0]
    relp_ref[...] = _ta_dot(_onehot_rows(r2_ref.shape[0], rl, F32),
                            r2_ref[...])[None]


def _layer_call(layer_kernel, x_operands, r, idx_ops, weights, shapes):
    n_ent, d_out, e_h, r2 = shapes
    x_specs = [pl.BlockSpec(op.shape, lambda d, n=op.ndim: (0,) * n)
               for op in x_operands]
    idx_specs = [pl.BlockSpec((1, e_h), lambda d: (0, d)) for _ in idx_ops]
    w_specs = [pl.BlockSpec(w.shape, lambda d, n=w.ndim: (0,) * n)
               for w in weights]
    return pl.pallas_call(
        layer_kernel,
        grid=(2,),
        in_specs=x_specs
        + [pl.BlockSpec((r2 // 2, r.shape[1]), lambda d: (d, 0))]
        + idx_specs + w_specs,
        out_specs=(
            pl.BlockSpec((1, n_ent, d_out), lambda d: (d, 0, 0)),
            pl.BlockSpec((n_ent // 2, d_out), lambda d: (d, 0)),
            pl.BlockSpec((r2 // 2, d_out), lambda d: (d, 0)),
        ),
        out_shape=(
            jax.ShapeDtypeStruct((2, n_ent, d_out), BF16),
            jax.ShapeDtypeStruct((n_ent, d_out), BF16),
            jax.ShapeDtypeStruct((r2, d_out), F32),
        ),
        compiler_params=pltpu.CompilerParams(
            dimension_semantics=("arbitrary",)),
    )(*x_operands, r, *idx_ops, *weights)


def _affine(bias, gamma, beta, mean, var, d_out, eps=1e-5):
    scale = gamma * jax.lax.rsqrt(var + eps)
    shift = (bias - mean) * scale + beta
    return (scale * (1.0 / 3.0)).reshape(1, d_out), shift.reshape(1, d_out)


def kernel(init_embed, init_rel, l0_in_w, l0_out_w, l0_loop_w, l0_w_rel,
           l0_loop_rel, l0_bias, l0_bn_gamma, l0_bn_beta, l0_bn_mean,
           l0_bn_var, l1_in_w, l1_out_w, l1_loop_w, l1_w_rel, l1_loop_rel,
           l1_bias, l1_bn_gamma, l1_bn_beta, l1_bn_mean, l1_bn_var,
           src, dst, etype, norm, subj, rel):
    n_ent, d_in = init_embed.shape
    r2 = init_rel.shape[0]
    e2 = src.shape[0]
    e_h = e2 // 2
    d_out = l0_in_w.shape[1]
    batch = subj.shape[0]
    shapes = (n_ent, d_out, e_h, r2)

    srcr = src.reshape(1, e2).astype(jnp.int32)
    dstr = dst.reshape(1, e2).astype(jnp.int32)
    etr = etype.reshape(1, e2).astype(jnp.int32)
    normr = norm.reshape(1, e2).astype(F32)
    idx_ops = (srcr, dstr, etr, normr)

    scale0, shift0 = _affine(l0_bias, l0_bn_gamma, l0_bn_beta, l0_bn_mean,
                             l0_bn_var, d_out)
    scale1, shift1 = _affine(l1_bias, l1_bn_gamma, l1_bn_beta, l1_bn_mean,
                             l1_bn_var, d_out)

    agg0, loop0, r1 = _layer_call(
        _layer0_kernel, (init_embed,), init_rel, idx_ops,
        (l0_in_w, l0_out_w, l0_loop_w, l0_w_rel, l0_loop_rel), shapes)

    agg1, loop1, r2_arr = _layer_call(
        _layer1_kernel, (agg0, loop0, scale0, shift0), r1, idx_ops,
        (l1_in_w, l1_out_w, l1_loop_w, l1_w_rel, l1_loop_rel), shapes)

    subjr = subj.reshape(1, batch).astype(jnp.int32)
    relr = rel.reshape(1, batch).astype(jnp.int32)
    full = lambda a: pl.BlockSpec(a.shape, lambda d: (0,) * a.ndim)
    x2, subp, relp = pl.pallas_call(
        _final_kernel,
        grid=(2,),
        in_specs=[
            pl.BlockSpec((2, n_ent // 2, d_out), lambda d: (0, d, 0)),
            pl.BlockSpec((n_ent // 2, d_out), lambda d: (d, 0)),
            full(scale1), full(shift1),
            pl.BlockSpec((r2 // 2, d_out), lambda d: (d, 0)),
            full(subjr), full(relr),
        ],
        out_specs=(
            pl.BlockSpec((n_ent // 2, d_out), lambda d: (d, 0)),
            pl.BlockSpec((1, batch, d_out), lambda d: (d, 0, 0)),
            pl.BlockSpec((1, batch, d_out), lambda d: (d, 0, 0)),
        ),
        out_shape=(
            jax.ShapeDtypeStruct((n_ent, d_out), F32),
            jax.ShapeDtypeStruct((2, batch, d_out), F32),
            jax.ShapeDtypeStruct((2, batch, d_out), F32),
        ),
        compiler_params=pltpu.CompilerParams(
            dimension_semantics=("arbitrary",)),
    )(agg1, loop1, scale1, shift1, r2_arr, subjr, relr)

    sub_emb = subp[0] + subp[1]
    rel_emb = relp[0] + relp[1]
    return sub_emb, rel_emb, x2
